# baseline (speedup 1.0000x reference)
import jax
import jax.numpy as jnp
from jax import lax
from jax.experimental import pallas as pl
from jax.experimental.pallas import tpu as pltpu, tpu_sc as plsc

VOCAB = 100000
DIM = 1024
NPOS = 2048
BATCH = 4

NC = 2
NS = 16
NW = NC * NS
LANES = 16

POS_PER_W = NPOS // NW
CP = 16
NCHUNK = POS_PER_W // CP
NSTEP = NCHUNK * BATCH
NBUF = 4
DVEC = DIM // LANES


def _body(tok_hbm, w_hbm, pos_hbm, out_hbm,
          idx_v, p0v, p1v, r0, r1, r2, r3,
          pg0, pg1, g0, g1, g2, g3, s0, s1, s2, s3):
    wid = lax.axis_index("s") * NC + lax.axis_index("c")
    p_base = wid * POS_PER_W
    rows = (r0, r1, r2, r3)
    posb = (p0v, p1v)
    psem = (pg0, pg1)
    gsem = (g0, g1, g2, g3)
    ssem = (s0, s1, s2, s3)

    for b in range(BATCH):
        pltpu.sync_copy(
            tok_hbm.at[pl.ds(b * NPOS + p_base, POS_PER_W)],
            idx_v.at[pl.ds(b * POS_PER_W, POS_PER_W)],
        )

    def load_pos(c):
        return pltpu.async_copy(
            pos_hbm.at[pl.ds(p_base + c * CP, CP)], posb[c % 2], psem[c % 2]
        )

    def gather(k):
        c, b = divmod(k, BATCH)
        return pltpu.async_copy(
            w_hbm.at[idx_v.at[pl.ds(b * POS_PER_W + c * CP, CP)]],
            rows[k % NBUF], gsem[k % NBUF],
        )

    pos_pending = load_pos(0)
    gathers = [gather(k) for k in range(NBUF - 1)] + [None]
    stores = [None] * NBUF

    for k in range(NSTEP):
        buf = k % NBUF
        c, b = divmod(k, BATCH)
        if b == 0:
            pos_pending.wait()
            if c + 1 < NCHUNK:
                pos_pending = load_pos(c + 1)
        gathers[buf].wait()
        if k + NBUF - 1 < NSTEP:
            nb = (k + NBUF - 1) % NBUF
            if stores[nb] is not None:
                stores[nb].wait()
                stores[nb] = None
            gathers[nb] = gather(k + NBUF - 1)

        pv = posb[c % 2]
        rv = rows[buf]

        @pl.loop(0, CP)
        def _row(r):
            for d in range(DVEC):
                off = d * LANES
                plsc.addupdate(
                    rv.at[r, pl.ds(off, LANES)],
                    pv[r, pl.ds(off, LANES)],
                )

        stores[buf] = pltpu.async_copy(
            rv,
            out_hbm.at[pl.ds(b * NPOS + p_base + c * CP, CP)],
            ssem[buf],
        )
    for st in stores:
        if st is not None:
            st.wait()


@jax.jit
def _embed(tokens, W, pos_enc):
    mesh = plsc.VectorSubcoreMesh(
        core_axis_name="c", subcore_axis_name="s",
        num_cores=NC, num_subcores=NS,
    )
    run = pl.kernel(
        _body,
        out_type=jax.ShapeDtypeStruct((BATCH * NPOS, DIM), jnp.float32),
        mesh=mesh,
        scratch_types=[
            pltpu.VMEM((BATCH * POS_PER_W,), jnp.int32),
            pltpu.VMEM((CP, DIM), jnp.float32),
            pltpu.VMEM((CP, DIM), jnp.float32),
            pltpu.VMEM((CP, DIM), jnp.float32),
            pltpu.VMEM((CP, DIM), jnp.float32),
            pltpu.VMEM((CP, DIM), jnp.float32),
            pltpu.VMEM((CP, DIM), jnp.float32),
            pltpu.SemaphoreType.DMA,
            pltpu.SemaphoreType.DMA,
            pltpu.SemaphoreType.DMA,
            pltpu.SemaphoreType.DMA,
            pltpu.SemaphoreType.DMA,
            pltpu.SemaphoreType.DMA,
            pltpu.SemaphoreType.DMA,
            pltpu.SemaphoreType.DMA,
            pltpu.SemaphoreType.DMA,
            pltpu.SemaphoreType.DMA,
        ],
    )
    out = run(tokens.reshape(-1), W, pos_enc)
    return out.reshape(BATCH, NPOS, DIM)


def kernel(tokens, W, pos_enc):
    return _embed(tokens.astype(jnp.int32), W, pos_enc)


# speedup vs baseline: 1.0513x; 1.0513x over previous
import jax
import jax.numpy as jnp
from jax import lax
from jax.experimental import pallas as pl
from jax.experimental.pallas import tpu as pltpu, tpu_sc as plsc

VOCAB = 100000
DIM = 1024
NPOS = 2048
BATCH = 4

NC = 2
NS = 16
NW = NC * NS
LANES = 16

POS_PER_W = NPOS // NW
CP = 8
NSTEP = POS_PER_W // CP
DVEC = DIM // LANES


def _body(tok_hbm, w_hbm, pos_hbm, out_hbm,
          idx_v, p0v, p1v,
          r00, r01, r02, r03, r10, r11, r12, r13,
          pg0, pg1,
          g00, g01, g02, g03, g10, g11, g12, g13,
          s00, s01, s02, s03, s10, s11, s12, s13):
    wid = lax.axis_index("s") * NC + lax.axis_index("c")
    p_base = wid * POS_PER_W
    rows = ((r00, r01, r02, r03), (r10, r11, r12, r13))
    posb = (p0v, p1v)
    psem = (pg0, pg1)
    gsem = ((g00, g01, g02, g03), (g10, g11, g12, g13))
    ssem = ((s00, s01, s02, s03), (s10, s11, s12, s13))

    for b in range(BATCH):
        pltpu.sync_copy(
            tok_hbm.at[pl.ds(b * NPOS + p_base, POS_PER_W)],
            idx_v.at[pl.ds(b * POS_PER_W, POS_PER_W)],
        )

    def load_pos(c):
        return pltpu.async_copy(
            pos_hbm.at[pl.ds(p_base + c * CP, CP)], posb[c % 2], psem[c % 2]
        )

    def gather(c, b):
        q = c % 2
        return pltpu.async_copy(
            w_hbm.at[idx_v.at[pl.ds(b * POS_PER_W + c * CP, CP)]],
            rows[q][b], gsem[q][b],
        )

    pos_pending = load_pos(0)
    gathers = [[None] * BATCH, [None] * BATCH]
    stores = [[None] * BATCH, [None] * BATCH]
    for b in range(BATCH):
        gathers[0][b] = gather(0, b)

    for c in range(NSTEP):
        q = c % 2
        nq = 1 - q
        pos_pending.wait()
        if c + 1 < NSTEP:
            pos_pending = load_pos(c + 1)
            for b in range(BATCH):
                if stores[nq][b] is not None:
                    stores[nq][b].wait()
                    stores[nq][b] = None
                gathers[nq][b] = gather(c + 1, b)
        for b in range(BATCH):
            gathers[q][b].wait()

        pv = posb[q]
        rbufs = rows[q]

        @pl.loop(0, CP)
        def _row(r):
            for d in range(DVEC):
                off = d * LANES
                pvec = pv[r, pl.ds(off, LANES)]
                for b in range(BATCH):
                    plsc.addupdate(rbufs[b].at[r, pl.ds(off, LANES)], pvec)

        for b in range(BATCH):
            stores[q][b] = pltpu.async_copy(
                rbufs[b],
                out_hbm.at[pl.ds(b * NPOS + p_base + c * CP, CP)],
                ssem[q][b],
            )
    for q in range(2):
        for b in range(BATCH):
            if stores[q][b] is not None:
                stores[q][b].wait()


@jax.jit
def _embed(tokens, W, pos_enc):
    mesh = plsc.VectorSubcoreMesh(
        core_axis_name="c", subcore_axis_name="s",
        num_cores=NC, num_subcores=NS,
    )
    run = pl.kernel(
        _body,
        out_type=jax.ShapeDtypeStruct((BATCH * NPOS, DIM), jnp.float32),
        mesh=mesh,
        scratch_types=[
            pltpu.VMEM((BATCH * POS_PER_W,), jnp.int32),
            pltpu.VMEM((CP, DIM), jnp.float32),
            pltpu.VMEM((CP, DIM), jnp.float32),
        ] + [pltpu.VMEM((CP, DIM), jnp.float32)] * 8
          + [pltpu.SemaphoreType.DMA] * 18,
    )
    out = run(tokens.reshape(-1), W, pos_enc)
    return out.reshape(BATCH, NPOS, DIM)


def kernel(tokens, W, pos_enc):
    return _embed(tokens.astype(jnp.int32), W, pos_enc)


# speedup vs baseline: 1.0717x; 1.0193x over previous
import jax
import jax.numpy as jnp
from jax import lax
from jax.experimental import pallas as pl
from jax.experimental.pallas import tpu as pltpu, tpu_sc as plsc

VOCAB = 100000
DIM = 1024
NPOS = 2048
BATCH = 4

NC = 2
NS = 16
NW = NC * NS
LANES = 16

POS_PER_W = NPOS // NW
CP = 8
NSTEP = POS_PER_W // CP
DVEC = DIM // LANES


def _body(tok_hbm, w_hbm, pos_hbm, out_hbm,
          idx_v, p0v, p1v,
          r00, r01, r02, r03, r10, r11, r12, r13,
          pg0, pg1,
          g00, g01, g02, g03, g10, g11, g12, g13,
          s00, s01, s02, s03, s10, s11, s12, s13):
    wid = lax.axis_index("s") * NC + lax.axis_index("c")
    p_base = wid * POS_PER_W
    rows = ((r00, r01, r02, r03), (r10, r11, r12, r13))
    posb = (p0v, p1v)
    psem = (pg0, pg1)
    gsem = ((g00, g01, g02, g03), (g10, g11, g12, g13))
    ssem = ((s00, s01, s02, s03), (s10, s11, s12, s13))

    idx_copies = [
        pltpu.async_copy(
            tok_hbm.at[pl.ds(b * NPOS + p_base, POS_PER_W)],
            idx_v.at[pl.ds(b * POS_PER_W, POS_PER_W)],
            ssem[1][b],
        )
        for b in range(BATCH)
    ]

    def load_pos(c):
        return pltpu.async_copy(
            pos_hbm.at[pl.ds(p_base + c * CP, CP)], posb[c % 2], psem[c % 2]
        )

    def gather(c, b):
        q = c % 2
        return pltpu.async_copy(
            w_hbm.at[idx_v.at[pl.ds(b * POS_PER_W + c * CP, CP)]],
            rows[q][b], gsem[q][b],
        )

    pos_pending = load_pos(0)
    gathers = [[None] * BATCH, [None] * BATCH]
    stores = [[None] * BATCH, [None] * BATCH]
    for b in range(BATCH):
        idx_copies[b].wait()
        gathers[0][b] = gather(0, b)

    for c in range(NSTEP):
        q = c % 2
        nq = 1 - q
        pos_pending.wait()
        if c + 1 < NSTEP:
            pos_pending = load_pos(c + 1)
            for b in range(BATCH):
                if stores[nq][b] is not None:
                    stores[nq][b].wait()
                    stores[nq][b] = None
                gathers[nq][b] = gather(c + 1, b)
        for b in range(BATCH):
            gathers[q][b].wait()

        pv = posb[q]
        rbufs = rows[q]

        @pl.loop(0, CP)
        def _row(r):
            for d in range(DVEC):
                off = d * LANES
                pvec = pv[r, pl.ds(off, LANES)]
                for b in range(BATCH):
                    plsc.addupdate(rbufs[b].at[r, pl.ds(off, LANES)], pvec)

        for b in range(BATCH):
            stores[q][b] = pltpu.async_copy(
                rbufs[b],
                out_hbm.at[pl.ds(b * NPOS + p_base + c * CP, CP)],
                ssem[q][b],
            )
    for q in range(2):
        for b in range(BATCH):
            if stores[q][b] is not None:
                stores[q][b].wait()


@jax.jit
def _embed(tokens, W, pos_enc):
    mesh = plsc.VectorSubcoreMesh(
        core_axis_name="c", subcore_axis_name="s",
        num_cores=NC, num_subcores=NS,
    )
    run = pl.kernel(
        _body,
        out_type=jax.ShapeDtypeStruct((BATCH * NPOS, DIM), jnp.float32),
        mesh=mesh,
        scratch_types=[
            pltpu.VMEM((BATCH * POS_PER_W,), jnp.int32),
            pltpu.VMEM((CP, DIM), jnp.float32),
            pltpu.VMEM((CP, DIM), jnp.float32),
        ] + [pltpu.VMEM((CP, DIM), jnp.float32)] * 8
          + [pltpu.SemaphoreType.DMA] * 18,
    )
    out = run(tokens.reshape(-1), W, pos_enc)
    return out.reshape(BATCH, NPOS, DIM)


def kernel(tokens, W, pos_enc):
    return _embed(tokens.astype(jnp.int32), W, pos_enc)
